# baseline (speedup 1.0000x reference)
import functools

import jax
import jax.numpy as jnp
from jax import lax
from jax.experimental import pallas as pl
from jax.experimental.pallas import tpu as pltpu
from jax.experimental.pallas import tpu_sc as plsc

N, E, NFEAT, NHID = 10000, 320000, 128, 128
BN_EPS = 1e-5
SN_EPS = 1e-12

NC, NS = 2, 16
NW = NC * NS
CHUNK = 128
NCHUNK = 80
EPT = NCHUNK * CHUNK
E_PAD = EPT * NW
NPAIR = NCHUNK // 2
N_SC = 10240
RPT = N_SC // NS


def _sc_aggregate_body(x_hbm, src_hbm, dst_hbm, zeros_hbm, out_hbm,
                       srcv, dstv, rows, agg, sem):
    c = lax.axis_index("c")
    s = lax.axis_index("s")
    row0 = s * RPT

    pltpu.sync_copy(zeros_hbm, agg.at[pl.ds(row0, RPT)])
    plsc.subcore_barrier()

    wid = s * NC + c

    def body(j, carry):
        pltpu.sync_copy(src_hbm.at[wid, j], srcv)
        pltpu.sync_copy(dst_hbm.at[wid, j], dstv)
        pltpu.async_copy(x_hbm.at[srcv], rows, sem).wait()
        pltpu.sync_copy(rows, agg.at[dstv], add=True)
        return carry

    lax.fori_loop(0, NCHUNK, body, 0)
    plsc.subcore_barrier()

    pltpu.sync_copy(agg.at[pl.ds(row0, RPT)],
                    out_hbm.at[c, pl.ds(row0, RPT)])


@functools.cache
def _get_sc_aggregate():
    mesh = plsc.VectorSubcoreMesh(core_axis_name="c", subcore_axis_name="s",
                                  num_cores=NC, num_subcores=NS)
    return pl.kernel(
        _sc_aggregate_body,
        out_type=jax.ShapeDtypeStruct((NC, N_SC, NFEAT), jnp.float32),
        mesh=mesh,
        scratch_types=[
            pltpu.VMEM((CHUNK,), jnp.int32),
            pltpu.VMEM((CHUNK,), jnp.int32),
            pltpu.VMEM((CHUNK, NFEAT), jnp.float32),
            pltpu.VMEM_SHARED((N_SC, NFEAT), jnp.float32),
            pltpu.SemaphoreType.DMA,
        ],
    )


_PREC = lax.Precision.HIGHEST


def _spectral(W, u):
    vT = lax.dot_general(u, W, (((1,), (0,)), ((), ())),
                         precision=_PREC)
    v = vT / (jnp.sqrt(jnp.sum(vT * vT)) + SN_EPS)
    Wv = lax.dot_general(v, W, (((1,), (1,)), ((), ())),
                         precision=_PREC)
    u_new = Wv / (jnp.sqrt(jnp.sum(Wv * Wv)) + SN_EPS)
    sigma = jnp.sum(u_new * Wv)
    return W * (1.0 / sigma)


BLK = 2000
GRID = N // BLK


def _tc1_body(x_ref, agg_ref, w1_ref, b1_ref, u1_ref, h_ref, s_ref, q_ref):
    i = pl.program_id(0)
    W1n = _spectral(w1_ref[...], u1_ref[...])
    h0 = x_ref[...] + agg_ref[0] + agg_ref[1]
    h = lax.dot_general(h0, W1n, (((1,), (1,)), ((), ())), precision=_PREC)
    h = jnp.maximum(h + b1_ref[...], 0.0)
    h_ref[...] = h

    @pl.when(i == 0)
    def _():
        s_ref[...] = jnp.zeros_like(s_ref)
        q_ref[...] = jnp.zeros_like(q_ref)

    s_ref[...] += jnp.sum(h, axis=0, keepdims=True)
    q_ref[...] += jnp.sum(h * h, axis=0, keepdims=True)


_tc1 = pl.pallas_call(
    _tc1_body,
    grid=(GRID,),
    in_specs=[
        pl.BlockSpec((BLK, NFEAT), lambda i: (i, 0)),
        pl.BlockSpec((NC, BLK, NFEAT), lambda i: (0, i, 0)),
        pl.BlockSpec((NHID, NFEAT), lambda i: (0, 0)),
        pl.BlockSpec((1, NHID), lambda i: (0, 0)),
        pl.BlockSpec((1, NHID), lambda i: (0, 0)),
    ],
    out_specs=[
        pl.BlockSpec((BLK, NHID), lambda i: (i, 0)),
        pl.BlockSpec((1, NHID), lambda i: (0, 0)),
        pl.BlockSpec((1, NHID), lambda i: (0, 0)),
    ],
    out_shape=[
        jax.ShapeDtypeStruct((N, NHID), jnp.float32),
        jax.ShapeDtypeStruct((1, NHID), jnp.float32),
        jax.ShapeDtypeStruct((1, NHID), jnp.float32),
    ],
)


def _tc2_body(h_ref, s_ref, q_ref, g_ref, be_ref, w2_ref, b2_ref, u2_ref,
              out_ref):
    W2n = _spectral(w2_ref[...], u2_ref[...])
    mean = s_ref[...] * (1.0 / N)
    var = q_ref[...] * (1.0 / N) - mean * mean
    sc = g_ref[...] * lax.rsqrt(var + BN_EPS)
    W2eff = W2n * sc
    cvec = lax.dot_general(be_ref[...] - mean * sc, W2n,
                           (((1,), (1,)), ((), ())), precision=_PREC)
    cvec = cvec + b2_ref[...]
    out = lax.dot_general(h_ref[...], W2eff, (((1,), (1,)), ((), ())),
                          precision=_PREC)
    out_ref[...] = out + cvec


_tc2 = pl.pallas_call(
    _tc2_body,
    grid=(GRID,),
    in_specs=[
        pl.BlockSpec((BLK, NHID), lambda i: (i, 0)),
        pl.BlockSpec((1, NHID), lambda i: (0, 0)),
        pl.BlockSpec((1, NHID), lambda i: (0, 0)),
        pl.BlockSpec((1, NHID), lambda i: (0, 0)),
        pl.BlockSpec((1, NHID), lambda i: (0, 0)),
        pl.BlockSpec((NHID, NHID), lambda i: (0, 0)),
        pl.BlockSpec((1, NHID), lambda i: (0, 0)),
        pl.BlockSpec((1, NHID), lambda i: (0, 0)),
    ],
    out_specs=pl.BlockSpec((BLK, NHID), lambda i: (i, 0)),
    out_shape=jax.ShapeDtypeStruct((N, NHID), jnp.float32),
)


def kernel(x, edge_index, W1, b1, gamma, beta, W2, b2, u1, u2):
    src = edge_index[0]
    dst = edge_index[1]
    pad = E_PAD - E
    src_p = jnp.concatenate([src, jnp.full((pad,), N, jnp.int32)])
    dst_p = jnp.concatenate([dst, jnp.zeros((pad,), jnp.int32)])
    src_p = src_p.reshape(NW, NCHUNK, CHUNK)
    dst_p = dst_p.reshape(NW, NCHUNK, CHUNK)
    x_p = jnp.concatenate([x, jnp.zeros((N_SC - N, NFEAT), x.dtype)], axis=0)
    zeros_seed = jnp.zeros((RPT, NFEAT), jnp.float32)

    aggs = _get_sc_aggregate()(x_p, src_p, dst_p, zeros_seed)

    relu_h, sums, sumsq = _tc1(x, aggs, W1, b1.reshape(1, -1),
                               u1.reshape(1, -1))
    out = _tc2(relu_h, sums, sumsq, gamma.reshape(1, -1), beta.reshape(1, -1),
               W2, b2.reshape(1, -1), u2.reshape(1, -1))
    return out


# speedup vs baseline: 1.0007x; 1.0007x over previous
import functools

import jax
import jax.numpy as jnp
from jax import lax
from jax.experimental import pallas as pl
from jax.experimental.pallas import tpu as pltpu
from jax.experimental.pallas import tpu_sc as plsc

N, E, NFEAT, NHID = 10000, 320000, 128, 128
BN_EPS = 1e-5
SN_EPS = 1e-12

NC, NS = 2, 16
NW = NC * NS
CHUNK = 128
NCHUNK = 80
EPT = NCHUNK * CHUNK
E_PAD = EPT * NW
NPAIR = NCHUNK // 2
N_SC = 10240
RPT = N_SC // NS


def _sc_aggregate_body(x_hbm, src_hbm, dst_hbm, zeros_hbm, out_hbm,
                       srcv, dstv, rows, agg, sem):
    c = lax.axis_index("c")
    s = lax.axis_index("s")
    row0 = s * RPT

    pltpu.sync_copy(zeros_hbm, agg.at[pl.ds(row0, RPT)])
    plsc.subcore_barrier()

    wid = s * NC + c
    ebase = wid * EPT

    def body(j, carry):
        off = pl.multiple_of(ebase + j * CHUNK, CHUNK)
        pltpu.sync_copy(src_hbm.at[pl.ds(off, CHUNK)], srcv)
        pltpu.sync_copy(dst_hbm.at[pl.ds(off, CHUNK)], dstv)
        pltpu.async_copy(x_hbm.at[srcv], rows, sem).wait()
        pltpu.sync_copy(rows, agg.at[dstv], add=True)
        return carry

    lax.fori_loop(0, NCHUNK, body, 0)
    plsc.subcore_barrier()

    pltpu.sync_copy(agg.at[pl.ds(row0, RPT)],
                    out_hbm.at[c, pl.ds(row0, RPT)])


@functools.cache
def _get_sc_aggregate():
    mesh = plsc.VectorSubcoreMesh(core_axis_name="c", subcore_axis_name="s",
                                  num_cores=NC, num_subcores=NS)
    return pl.kernel(
        _sc_aggregate_body,
        out_type=jax.ShapeDtypeStruct((NC, N_SC, NFEAT), jnp.float32),
        mesh=mesh,
        scratch_types=[
            pltpu.VMEM((CHUNK,), jnp.int32),
            pltpu.VMEM((CHUNK,), jnp.int32),
            pltpu.VMEM((CHUNK, NFEAT), jnp.float32),
            pltpu.VMEM_SHARED((N_SC, NFEAT), jnp.float32),
            pltpu.SemaphoreType.DMA,
        ],
    )


_PREC = lax.Precision.HIGHEST


def _spectral(W, u):
    vT = lax.dot_general(u, W, (((1,), (0,)), ((), ())),
                         precision=_PREC)
    v = vT / (jnp.sqrt(jnp.sum(vT * vT)) + SN_EPS)
    Wv = lax.dot_general(v, W, (((1,), (1,)), ((), ())),
                         precision=_PREC)
    u_new = Wv / (jnp.sqrt(jnp.sum(Wv * Wv)) + SN_EPS)
    sigma = jnp.sum(u_new * Wv)
    return W * (1.0 / sigma)


BLK = 2000
GRID = N // BLK


def _tc1_body(x_ref, agg_ref, w1_ref, b1_ref, u1_ref, h_ref, s_ref, q_ref):
    i = pl.program_id(0)
    W1n = _spectral(w1_ref[...], u1_ref[...])
    h0 = x_ref[...] + agg_ref[0] + agg_ref[1]
    h = lax.dot_general(h0, W1n, (((1,), (1,)), ((), ())), precision=_PREC)
    h = jnp.maximum(h + b1_ref[...], 0.0)
    h_ref[...] = h

    @pl.when(i == 0)
    def _():
        s_ref[...] = jnp.zeros_like(s_ref)
        q_ref[...] = jnp.zeros_like(q_ref)

    s_ref[...] += jnp.sum(h, axis=0, keepdims=True)
    q_ref[...] += jnp.sum(h * h, axis=0, keepdims=True)


_tc1 = pl.pallas_call(
    _tc1_body,
    grid=(GRID,),
    in_specs=[
        pl.BlockSpec((BLK, NFEAT), lambda i: (i, 0)),
        pl.BlockSpec((NC, BLK, NFEAT), lambda i: (0, i, 0)),
        pl.BlockSpec((NHID, NFEAT), lambda i: (0, 0)),
        pl.BlockSpec((1, NHID), lambda i: (0, 0)),
        pl.BlockSpec((1, NHID), lambda i: (0, 0)),
    ],
    out_specs=[
        pl.BlockSpec((BLK, NHID), lambda i: (i, 0)),
        pl.BlockSpec((1, NHID), lambda i: (0, 0)),
        pl.BlockSpec((1, NHID), lambda i: (0, 0)),
    ],
    out_shape=[
        jax.ShapeDtypeStruct((N, NHID), jnp.float32),
        jax.ShapeDtypeStruct((1, NHID), jnp.float32),
        jax.ShapeDtypeStruct((1, NHID), jnp.float32),
    ],
)


def _tc2_body(h_ref, s_ref, q_ref, g_ref, be_ref, w2_ref, b2_ref, u2_ref,
              out_ref):
    W2n = _spectral(w2_ref[...], u2_ref[...])
    mean = s_ref[...] * (1.0 / N)
    var = q_ref[...] * (1.0 / N) - mean * mean
    sc = g_ref[...] * lax.rsqrt(var + BN_EPS)
    W2eff = W2n * sc
    cvec = lax.dot_general(be_ref[...] - mean * sc, W2n,
                           (((1,), (1,)), ((), ())), precision=_PREC)
    cvec = cvec + b2_ref[...]
    out = lax.dot_general(h_ref[...], W2eff, (((1,), (1,)), ((), ())),
                          precision=_PREC)
    out_ref[...] = out + cvec


_tc2 = pl.pallas_call(
    _tc2_body,
    grid=(GRID,),
    in_specs=[
        pl.BlockSpec((BLK, NHID), lambda i: (i, 0)),
        pl.BlockSpec((1, NHID), lambda i: (0, 0)),
        pl.BlockSpec((1, NHID), lambda i: (0, 0)),
        pl.BlockSpec((1, NHID), lambda i: (0, 0)),
        pl.BlockSpec((1, NHID), lambda i: (0, 0)),
        pl.BlockSpec((NHID, NHID), lambda i: (0, 0)),
        pl.BlockSpec((1, NHID), lambda i: (0, 0)),
        pl.BlockSpec((1, NHID), lambda i: (0, 0)),
    ],
    out_specs=pl.BlockSpec((BLK, NHID), lambda i: (i, 0)),
    out_shape=jax.ShapeDtypeStruct((N, NHID), jnp.float32),
)


def kernel(x, edge_index, W1, b1, gamma, beta, W2, b2, u1, u2):
    src = edge_index[0]
    dst = edge_index[1]
    pad = E_PAD - E
    src_p = jnp.concatenate([src, jnp.full((pad,), N, jnp.int32)])
    dst_p = jnp.concatenate([dst, jnp.zeros((pad,), jnp.int32)])
    x_p = jnp.concatenate([x, jnp.zeros((N_SC - N, NFEAT), x.dtype)], axis=0)
    zeros_seed = jnp.zeros((RPT, NFEAT), jnp.float32)

    aggs = _get_sc_aggregate()(x_p, src_p, dst_p, zeros_seed)

    relu_h, sums, sumsq = _tc1(x, aggs, W1, b1.reshape(1, -1),
                               u1.reshape(1, -1))
    out = _tc2(relu_h, sums, sumsq, gamma.reshape(1, -1), beta.reshape(1, -1),
               W2, b2.reshape(1, -1), u2.reshape(1, -1))
    return out


# speedup vs baseline: 1.0018x; 1.0011x over previous
import functools

import jax
import jax.numpy as jnp
from jax import lax
from jax.experimental import pallas as pl
from jax.experimental.pallas import tpu as pltpu
from jax.experimental.pallas import tpu_sc as plsc

N, E, NFEAT, NHID = 10000, 320000, 128, 128
BN_EPS = 1e-5
SN_EPS = 1e-12

NC, NS = 2, 16
NW = NC * NS
CHUNK = 128
NCHUNK = 80
EPT = NCHUNK * CHUNK
E_PAD = EPT * NW
NPAIR = NCHUNK // 2
N_SC = 10240
RPT = N_SC // NS


def _sc_aggregate_body(x_hbm, src_hbm, dst_hbm, out_hbm,
                       srcv, dstv, rows, agg, sem):
    c = lax.axis_index("c")
    s = lax.axis_index("s")
    row0 = s * RPT

    zvec = jnp.zeros((16,), jnp.float32)

    def zbody(i, carry):
        rows[i >> 3, pl.ds((i & 7) * 16, 16)] = zvec
        return carry

    lax.fori_loop(0, CHUNK * 8, zbody, 0)
    for kk in range(RPT // CHUNK):
        pltpu.sync_copy(rows, agg.at[pl.ds(row0 + kk * CHUNK, CHUNK)])
    plsc.subcore_barrier()

    wid = s * NC + c
    ebase = wid * EPT

    def body(j, carry):
        off = pl.multiple_of(ebase + j * CHUNK, CHUNK)
        pltpu.sync_copy(src_hbm.at[pl.ds(off, CHUNK)], srcv)
        pltpu.sync_copy(dst_hbm.at[pl.ds(off, CHUNK)], dstv)
        pltpu.async_copy(x_hbm.at[srcv], rows, sem).wait()
        pltpu.sync_copy(rows, agg.at[dstv], add=True)
        return carry

    lax.fori_loop(0, NCHUNK, body, 0)
    plsc.subcore_barrier()

    pltpu.sync_copy(agg.at[pl.ds(row0, RPT)],
                    out_hbm.at[c, pl.ds(row0, RPT)])


@functools.cache
def _get_sc_aggregate():
    mesh = plsc.VectorSubcoreMesh(core_axis_name="c", subcore_axis_name="s",
                                  num_cores=NC, num_subcores=NS)
    return pl.kernel(
        _sc_aggregate_body,
        out_type=jax.ShapeDtypeStruct((NC, N_SC, NFEAT), jnp.float32),
        mesh=mesh,
        scratch_types=[
            pltpu.VMEM((CHUNK,), jnp.int32),
            pltpu.VMEM((CHUNK,), jnp.int32),
            pltpu.VMEM((CHUNK, NFEAT), jnp.float32),
            pltpu.VMEM_SHARED((N_SC, NFEAT), jnp.float32),
            pltpu.SemaphoreType.DMA,
        ],
    )


_PREC = lax.Precision.HIGHEST


def _spectral(W, u):
    vT = lax.dot_general(u, W, (((1,), (0,)), ((), ())),
                         precision=_PREC)
    v = vT / (jnp.sqrt(jnp.sum(vT * vT)) + SN_EPS)
    Wv = lax.dot_general(v, W, (((1,), (1,)), ((), ())),
                         precision=_PREC)
    u_new = Wv / (jnp.sqrt(jnp.sum(Wv * Wv)) + SN_EPS)
    sigma = jnp.sum(u_new * Wv)
    return W * (1.0 / sigma)


BLK = 2000
GRID = N // BLK


def _tc1_body(x_ref, agg_ref, w1_ref, b1_ref, u1_ref, h_ref, s_ref, q_ref):
    i = pl.program_id(0)
    W1n = _spectral(w1_ref[...], u1_ref[...])
    h0 = x_ref[...] + agg_ref[0] + agg_ref[1]
    h = lax.dot_general(h0, W1n, (((1,), (1,)), ((), ())), precision=_PREC)
    h = jnp.maximum(h + b1_ref[...], 0.0)
    h_ref[...] = h

    @pl.when(i == 0)
    def _():
        s_ref[...] = jnp.zeros_like(s_ref)
        q_ref[...] = jnp.zeros_like(q_ref)

    s_ref[...] += jnp.sum(h, axis=0, keepdims=True)
    q_ref[...] += jnp.sum(h * h, axis=0, keepdims=True)


_tc1 = pl.pallas_call(
    _tc1_body,
    grid=(GRID,),
    in_specs=[
        pl.BlockSpec((BLK, NFEAT), lambda i: (i, 0)),
        pl.BlockSpec((NC, BLK, NFEAT), lambda i: (0, i, 0)),
        pl.BlockSpec((NHID, NFEAT), lambda i: (0, 0)),
        pl.BlockSpec((1, NHID), lambda i: (0, 0)),
        pl.BlockSpec((1, NHID), lambda i: (0, 0)),
    ],
    out_specs=[
        pl.BlockSpec((BLK, NHID), lambda i: (i, 0)),
        pl.BlockSpec((1, NHID), lambda i: (0, 0)),
        pl.BlockSpec((1, NHID), lambda i: (0, 0)),
    ],
    out_shape=[
        jax.ShapeDtypeStruct((N, NHID), jnp.float32),
        jax.ShapeDtypeStruct((1, NHID), jnp.float32),
        jax.ShapeDtypeStruct((1, NHID), jnp.float32),
    ],
)


def _tc2_body(h_ref, s_ref, q_ref, g_ref, be_ref, w2_ref, b2_ref, u2_ref,
              out_ref):
    W2n = _spectral(w2_ref[...], u2_ref[...])
    mean = s_ref[...] * (1.0 / N)
    var = q_ref[...] * (1.0 / N) - mean * mean
    sc = g_ref[...] * lax.rsqrt(var + BN_EPS)
    W2eff = W2n * sc
    cvec = lax.dot_general(be_ref[...] - mean * sc, W2n,
                           (((1,), (1,)), ((), ())), precision=_PREC)
    cvec = cvec + b2_ref[...]
    out = lax.dot_general(h_ref[...], W2eff, (((1,), (1,)), ((), ())),
                          precision=_PREC)
    out_ref[...] = out + cvec


_tc2 = pl.pallas_call(
    _tc2_body,
    grid=(GRID,),
    in_specs=[
        pl.BlockSpec((BLK, NHID), lambda i: (i, 0)),
        pl.BlockSpec((1, NHID), lambda i: (0, 0)),
        pl.BlockSpec((1, NHID), lambda i: (0, 0)),
        pl.BlockSpec((1, NHID), lambda i: (0, 0)),
        pl.BlockSpec((1, NHID), lambda i: (0, 0)),
        pl.BlockSpec((NHID, NHID), lambda i: (0, 0)),
        pl.BlockSpec((1, NHID), lambda i: (0, 0)),
        pl.BlockSpec((1, NHID), lambda i: (0, 0)),
    ],
    out_specs=pl.BlockSpec((BLK, NHID), lambda i: (i, 0)),
    out_shape=jax.ShapeDtypeStruct((N, NHID), jnp.float32),
)


def kernel(x, edge_index, W1, b1, gamma, beta, W2, b2, u1, u2):
    src = edge_index[0]
    dst = edge_index[1]
    pad = E_PAD - E
    src_p = jnp.concatenate([src, jnp.full((pad,), N, jnp.int32)])
    dst_p = jnp.concatenate([dst, jnp.zeros((pad,), jnp.int32)])
    x_p = jnp.concatenate([x, jnp.zeros((N_SC - N, NFEAT), x.dtype)], axis=0)

    aggs = _get_sc_aggregate()(x_p, src_p, dst_p)

    relu_h, sums, sumsq = _tc1(x, aggs, W1, b1.reshape(1, -1),
                               u1.reshape(1, -1))
    out = _tc2(relu_h, sums, sumsq, gamma.reshape(1, -1), beta.reshape(1, -1),
               W2, b2.reshape(1, -1), u2.reshape(1, -1))
    return out


# speedup vs baseline: 1.0019x; 1.0001x over previous
import functools

import jax
import jax.numpy as jnp
from jax import lax
from jax.experimental import pallas as pl
from jax.experimental.pallas import tpu as pltpu
from jax.experimental.pallas import tpu_sc as plsc

N, E, NFEAT, NHID = 10000, 320000, 128, 128
BN_EPS = 1e-5
SN_EPS = 1e-12

NC, NS = 2, 16
NW = NC * NS
CHUNK = 128
NCHUNK = 80
EPT = NCHUNK * CHUNK
E_PAD = EPT * NW
NPAIR = NCHUNK // 2
N_SC = 10240
RPT = N_SC // NS


def _sc_aggregate_body(x_hbm, src_hbm, dst_hbm, out_hbm,
                       srcv, dstv, rows, agg, sem):
    c = lax.axis_index("c")
    s = lax.axis_index("s")
    row0 = s * RPT

    zvec = jnp.zeros((16,), jnp.float32)

    def zbody(i, carry):
        rows[i >> 3, pl.ds((i & 7) * 16, 16)] = zvec
        return carry

    lax.fori_loop(0, CHUNK * 8, zbody, 0)
    for kk in range(RPT // CHUNK):
        pltpu.sync_copy(rows, agg.at[pl.ds(row0 + kk * CHUNK, CHUNK)])
    plsc.subcore_barrier()

    wid = s * NC + c
    ebase = wid * EPT

    def body(j, carry):
        off = pl.multiple_of(ebase + j * CHUNK, CHUNK)
        pltpu.sync_copy(src_hbm.at[pl.ds(off, CHUNK)], srcv)
        pltpu.sync_copy(dst_hbm.at[pl.ds(off, CHUNK)], dstv)
        pltpu.async_copy(x_hbm.at[srcv], rows, sem).wait()
        pltpu.sync_copy(rows, agg.at[dstv], add=True)
        return carry

    lax.fori_loop(0, NCHUNK, body, 0)
    plsc.subcore_barrier()

    pltpu.sync_copy(agg.at[pl.ds(row0, RPT)],
                    out_hbm.at[c, pl.ds(row0, RPT)])


@functools.cache
def _get_sc_aggregate():
    mesh = plsc.VectorSubcoreMesh(core_axis_name="c", subcore_axis_name="s",
                                  num_cores=NC, num_subcores=NS)
    return pl.kernel(
        _sc_aggregate_body,
        out_type=jax.ShapeDtypeStruct((NC, N_SC, NFEAT), jnp.float32),
        mesh=mesh,
        scratch_types=[
            pltpu.VMEM((CHUNK,), jnp.int32),
            pltpu.VMEM((CHUNK,), jnp.int32),
            pltpu.VMEM((CHUNK, NFEAT), jnp.float32),
            pltpu.VMEM_SHARED((N_SC, NFEAT), jnp.float32),
            pltpu.SemaphoreType.DMA,
        ],
    )


_PREC = lax.Precision.HIGHEST


def _spectral(W, u):
    vT = lax.dot_general(u, W, (((1,), (0,)), ((), ())),
                         precision=_PREC)
    v = vT / (jnp.sqrt(jnp.sum(vT * vT)) + SN_EPS)
    Wv = lax.dot_general(v, W, (((1,), (1,)), ((), ())),
                         precision=_PREC)
    u_new = Wv / (jnp.sqrt(jnp.sum(Wv * Wv)) + SN_EPS)
    sigma = jnp.sum(u_new * Wv)
    return W * (1.0 / sigma)


BLK = 2000
GRID = N // BLK


def _tc1_body(x_ref, agg_ref, w1_ref, b1_ref, u1_ref, h_ref, s_ref, q_ref):
    i = pl.program_id(0)
    W1n = _spectral(w1_ref[...], u1_ref[...])
    h0 = x_ref[...] + agg_ref[0] + agg_ref[1]
    h = lax.dot_general(h0, W1n, (((1,), (1,)), ((), ())), precision=_PREC)
    h = jnp.maximum(h + b1_ref[...], 0.0)
    h_ref[...] = h

    @pl.when(i == 0)
    def _():
        s_ref[...] = jnp.zeros_like(s_ref)
        q_ref[...] = jnp.zeros_like(q_ref)

    s_ref[...] += jnp.sum(h, axis=0, keepdims=True)
    q_ref[...] += jnp.sum(h * h, axis=0, keepdims=True)


_tc1 = pl.pallas_call(
    _tc1_body,
    grid=(GRID,),
    in_specs=[
        pl.BlockSpec((BLK, NFEAT), lambda i: (i, 0)),
        pl.BlockSpec((NC, BLK, NFEAT), lambda i: (0, i, 0)),
        pl.BlockSpec((NHID, NFEAT), lambda i: (0, 0)),
        pl.BlockSpec((1, NHID), lambda i: (0, 0)),
        pl.BlockSpec((1, NHID), lambda i: (0, 0)),
    ],
    out_specs=[
        pl.BlockSpec((BLK, NHID), lambda i: (i, 0)),
        pl.BlockSpec((1, NHID), lambda i: (0, 0)),
        pl.BlockSpec((1, NHID), lambda i: (0, 0)),
    ],
    out_shape=[
        jax.ShapeDtypeStruct((N, NHID), jnp.float32),
        jax.ShapeDtypeStruct((1, NHID), jnp.float32),
        jax.ShapeDtypeStruct((1, NHID), jnp.float32),
    ],
)


def _tc2_body(h_ref, s_ref, q_ref, g_ref, be_ref, w2_ref, b2_ref, u2_ref,
              out_ref):
    W2n = _spectral(w2_ref[...], u2_ref[...])
    mean = s_ref[...] * (1.0 / N)
    var = q_ref[...] * (1.0 / N) - mean * mean
    sc = g_ref[...] * lax.rsqrt(var + BN_EPS)
    W2eff = W2n * sc
    cvec = lax.dot_general(be_ref[...] - mean * sc, W2n,
                           (((1,), (1,)), ((), ())), precision=_PREC)
    cvec = cvec + b2_ref[...]
    out = lax.dot_general(h_ref[...], W2eff, (((1,), (1,)), ((), ())),
                          precision=_PREC)
    out_ref[...] = out + cvec


_tc2 = pl.pallas_call(
    _tc2_body,
    grid=(GRID,),
    in_specs=[
        pl.BlockSpec((BLK, NHID), lambda i: (i, 0)),
        pl.BlockSpec((1, NHID), lambda i: (0, 0)),
        pl.BlockSpec((1, NHID), lambda i: (0, 0)),
        pl.BlockSpec((1, NHID), lambda i: (0, 0)),
        pl.BlockSpec((1, NHID), lambda i: (0, 0)),
        pl.BlockSpec((NHID, NHID), lambda i: (0, 0)),
        pl.BlockSpec((1, NHID), lambda i: (0, 0)),
        pl.BlockSpec((1, NHID), lambda i: (0, 0)),
    ],
    out_specs=pl.BlockSpec((BLK, NHID), lambda i: (i, 0)),
    out_shape=jax.ShapeDtypeStruct((N, NHID), jnp.float32),
)


def kernel(x, edge_index, W1, b1, gamma, beta, W2, b2, u1, u2):
    src = edge_index[0]
    dst = edge_index[1]
    pad = E_PAD - E
    src_p = jnp.concatenate([src, jnp.full((pad,), N, jnp.int32)])
    pad_dst = N + jnp.arange(pad, dtype=jnp.int32) % (N_SC - N)
    dst_p = jnp.concatenate([dst, pad_dst])
    x_p = jnp.concatenate([x, jnp.zeros((N_SC - N, NFEAT), x.dtype)], axis=0)

    aggs = _get_sc_aggregate()(x_p, src_p, dst_p)

    relu_h, sums, sumsq = _tc1(x, aggs, W1, b1.reshape(1, -1),
                               u1.reshape(1, -1))
    out = _tc2(relu_h, sums, sumsq, gamma.reshape(1, -1), beta.reshape(1, -1),
               W2, b2.reshape(1, -1), u2.reshape(1, -1))
    return out


# speedup vs baseline: 2.3187x; 2.3143x over previous
import functools

import jax
import jax.numpy as jnp
from jax import lax
from jax.experimental import pallas as pl
from jax.experimental.pallas import tpu as pltpu
from jax.experimental.pallas import tpu_sc as plsc

N, E, NFEAT, NHID = 10000, 320000, 128, 128
BN_EPS = 1e-5
SN_EPS = 1e-12

NC, NS = 2, 16
NW = NC * NS
CHUNK = 128
EPT = E // NW
NFULL = EPT // CHUNK
TAIL = EPT - NFULL * CHUNK
N_SC = 10240
RPT = N_SC // NS


def _sc_aggregate_body(x_hbm, ei_hbm, out_hbm,
                       srcv, dstv, rows, srct, dstt, rowst, agg, sem):
    c = lax.axis_index("c")
    s = lax.axis_index("s")
    row0 = s * RPT

    zvec = jnp.zeros((16,), jnp.float32)

    def zbody(i, carry):
        rows[i >> 3, pl.ds((i & 7) * 16, 16)] = zvec
        return carry

    lax.fori_loop(0, CHUNK * 8, zbody, 0)
    for kk in range(RPT // CHUNK):
        pltpu.sync_copy(rows, agg.at[pl.ds(row0 + kk * CHUNK, CHUNK)])
    plsc.subcore_barrier()

    wid = s * NC + c
    ebase = wid * EPT

    def body(j, carry):
        off = pl.multiple_of(ebase + j * CHUNK, 16)
        pltpu.sync_copy(ei_hbm.at[pl.ds(off, CHUNK)], srcv)
        pltpu.sync_copy(ei_hbm.at[pl.ds(E + off, CHUNK)], dstv)
        pltpu.async_copy(x_hbm.at[srcv], rows, sem).wait()
        pltpu.sync_copy(rows, agg.at[dstv], add=True)
        return carry

    lax.fori_loop(0, NFULL, body, 0)

    offt = ebase + NFULL * CHUNK
    pltpu.sync_copy(ei_hbm.at[pl.ds(offt, TAIL)], srct)
    pltpu.sync_copy(ei_hbm.at[pl.ds(E + offt, TAIL)], dstt)
    pltpu.async_copy(x_hbm.at[srct], rowst, sem).wait()
    pltpu.sync_copy(rowst, agg.at[dstt], add=True)

    plsc.subcore_barrier()

    pltpu.sync_copy(agg.at[pl.ds(row0, RPT)],
                    out_hbm.at[c, pl.ds(row0, RPT)])


@functools.cache
def _get_sc_aggregate():
    mesh = plsc.VectorSubcoreMesh(core_axis_name="c", subcore_axis_name="s",
                                  num_cores=NC, num_subcores=NS)
    return pl.kernel(
        _sc_aggregate_body,
        out_type=jax.ShapeDtypeStruct((NC, N_SC, NFEAT), jnp.float32),
        mesh=mesh,
        scratch_types=[
            pltpu.VMEM((CHUNK,), jnp.int32),
            pltpu.VMEM((CHUNK,), jnp.int32),
            pltpu.VMEM((CHUNK, NFEAT), jnp.float32),
            pltpu.VMEM((TAIL,), jnp.int32),
            pltpu.VMEM((TAIL,), jnp.int32),
            pltpu.VMEM((TAIL, NFEAT), jnp.float32),
            pltpu.VMEM_SHARED((N_SC, NFEAT), jnp.float32),
            pltpu.SemaphoreType.DMA,
        ],
    )


_PREC = lax.Precision.HIGHEST


def _spectral(W, u):
    vT = lax.dot_general(u, W, (((1,), (0,)), ((), ())),
                         precision=_PREC)
    v = vT / (jnp.sqrt(jnp.sum(vT * vT)) + SN_EPS)
    Wv = lax.dot_general(v, W, (((1,), (1,)), ((), ())),
                         precision=_PREC)
    u_new = Wv / (jnp.sqrt(jnp.sum(Wv * Wv)) + SN_EPS)
    sigma = jnp.sum(u_new * Wv)
    return W * (1.0 / sigma)


BLK = 2000
GRID = N // BLK


def _tc1_body(x_ref, agg_ref, w1_ref, b1_ref, u1_ref, h_ref, s_ref, q_ref):
    i = pl.program_id(0)
    W1n = _spectral(w1_ref[...], u1_ref[...])
    h0 = x_ref[...] + agg_ref[0] + agg_ref[1]
    h = lax.dot_general(h0, W1n, (((1,), (1,)), ((), ())), precision=_PREC)
    h = jnp.maximum(h + b1_ref[...], 0.0)
    h_ref[...] = h

    @pl.when(i == 0)
    def _():
        s_ref[...] = jnp.zeros_like(s_ref)
        q_ref[...] = jnp.zeros_like(q_ref)

    s_ref[...] += jnp.sum(h, axis=0, keepdims=True)
    q_ref[...] += jnp.sum(h * h, axis=0, keepdims=True)


_tc1 = pl.pallas_call(
    _tc1_body,
    grid=(GRID,),
    in_specs=[
        pl.BlockSpec((BLK, NFEAT), lambda i: (i, 0)),
        pl.BlockSpec((NC, BLK, NFEAT), lambda i: (0, i, 0)),
        pl.BlockSpec((NHID, NFEAT), lambda i: (0, 0)),
        pl.BlockSpec((1, NHID), lambda i: (0, 0)),
        pl.BlockSpec((1, NHID), lambda i: (0, 0)),
    ],
    out_specs=[
        pl.BlockSpec((BLK, NHID), lambda i: (i, 0)),
        pl.BlockSpec((1, NHID), lambda i: (0, 0)),
        pl.BlockSpec((1, NHID), lambda i: (0, 0)),
    ],
    out_shape=[
        jax.ShapeDtypeStruct((N, NHID), jnp.float32),
        jax.ShapeDtypeStruct((1, NHID), jnp.float32),
        jax.ShapeDtypeStruct((1, NHID), jnp.float32),
    ],
)


def _tc2_body(h_ref, s_ref, q_ref, g_ref, be_ref, w2_ref, b2_ref, u2_ref,
              out_ref):
    W2n = _spectral(w2_ref[...], u2_ref[...])
    mean = s_ref[...] * (1.0 / N)
    var = q_ref[...] * (1.0 / N) - mean * mean
    sc = g_ref[...] * lax.rsqrt(var + BN_EPS)
    W2eff = W2n * sc
    cvec = lax.dot_general(be_ref[...] - mean * sc, W2n,
                           (((1,), (1,)), ((), ())), precision=_PREC)
    cvec = cvec + b2_ref[...]
    out = lax.dot_general(h_ref[...], W2eff, (((1,), (1,)), ((), ())),
                          precision=_PREC)
    out_ref[...] = out + cvec


_tc2 = pl.pallas_call(
    _tc2_body,
    grid=(GRID,),
    in_specs=[
        pl.BlockSpec((BLK, NHID), lambda i: (i, 0)),
        pl.BlockSpec((1, NHID), lambda i: (0, 0)),
        pl.BlockSpec((1, NHID), lambda i: (0, 0)),
        pl.BlockSpec((1, NHID), lambda i: (0, 0)),
        pl.BlockSpec((1, NHID), lambda i: (0, 0)),
        pl.BlockSpec((NHID, NHID), lambda i: (0, 0)),
        pl.BlockSpec((1, NHID), lambda i: (0, 0)),
        pl.BlockSpec((1, NHID), lambda i: (0, 0)),
    ],
    out_specs=pl.BlockSpec((BLK, NHID), lambda i: (i, 0)),
    out_shape=jax.ShapeDtypeStruct((N, NHID), jnp.float32),
)


def kernel(x, edge_index, W1, b1, gamma, beta, W2, b2, u1, u2):
    ei_flat = edge_index.reshape(2 * E)

    aggs = _get_sc_aggregate()(x, ei_flat)

    relu_h, sums, sumsq = _tc1(x, aggs, W1, b1.reshape(1, -1),
                               u1.reshape(1, -1))
    out = _tc2(relu_h, sums, sumsq, gamma.reshape(1, -1), beta.reshape(1, -1),
               W2, b2.reshape(1, -1), u2.reshape(1, -1))
    return out


# speedup vs baseline: 2.6327x; 1.1354x over previous
import functools

import jax
import jax.numpy as jnp
from jax import lax
from jax.experimental import pallas as pl
from jax.experimental.pallas import tpu as pltpu
from jax.experimental.pallas import tpu_sc as plsc

N, E, NFEAT, NHID = 10000, 320000, 128, 128
BN_EPS = 1e-5
SN_EPS = 1e-12

NC, NS = 2, 16
NW = NC * NS
CHUNK = 128
EPT = E // NW
NFULL = EPT // CHUNK
TAIL = EPT - NFULL * CHUNK
N_SC = 10240
RPT = N_SC // NS


def _sc_aggregate_body(x_hbm, ei_hbm, out_hbm,
                       srcall, dstv, rows, dstt, rowst, agg, sem):
    c = lax.axis_index("c")
    s = lax.axis_index("s")
    row0 = s * RPT

    zvec = jnp.zeros((16,), jnp.float32)

    def zbody(i, carry):
        rows[i >> 3, pl.ds((i & 7) * 16, 16)] = zvec
        return carry

    lax.fori_loop(0, CHUNK * 8, zbody, 0)
    for kk in range(RPT // CHUNK):
        pltpu.sync_copy(rows, agg.at[pl.ds(row0 + kk * CHUNK, CHUNK)])
    plsc.subcore_barrier()

    wid = s * NC + c
    ebase = wid * EPT

    pltpu.sync_copy(ei_hbm.at[pl.ds(pl.multiple_of(ebase, 16), EPT)], srcall)

    def body(j, carry):
        off = pl.multiple_of(ebase + j * CHUNK, 16)
        pltpu.sync_copy(ei_hbm.at[pl.ds(E + off, CHUNK)], dstv)
        pltpu.async_copy(x_hbm.at[srcall.at[pl.ds(j * CHUNK, CHUNK)]],
                         rows, sem).wait()
        pltpu.sync_copy(rows, agg.at[dstv], add=True)
        return carry

    lax.fori_loop(0, NFULL, body, 0)

    offt = ebase + NFULL * CHUNK
    pltpu.sync_copy(ei_hbm.at[pl.ds(E + offt, TAIL)], dstt)
    pltpu.async_copy(x_hbm.at[srcall.at[pl.ds(NFULL * CHUNK, TAIL)]],
                     rowst, sem).wait()
    pltpu.sync_copy(rowst, agg.at[dstt], add=True)

    plsc.subcore_barrier()

    pltpu.sync_copy(agg.at[pl.ds(row0, RPT)],
                    out_hbm.at[c, pl.ds(row0, RPT)])


@functools.cache
def _get_sc_aggregate():
    mesh = plsc.VectorSubcoreMesh(core_axis_name="c", subcore_axis_name="s",
                                  num_cores=NC, num_subcores=NS)
    return pl.kernel(
        _sc_aggregate_body,
        out_type=jax.ShapeDtypeStruct((NC, N_SC, NFEAT), jnp.float32),
        mesh=mesh,
        scratch_types=[
            pltpu.VMEM((EPT,), jnp.int32),
            pltpu.VMEM((CHUNK,), jnp.int32),
            pltpu.VMEM((CHUNK, NFEAT), jnp.float32),
            pltpu.VMEM((TAIL,), jnp.int32),
            pltpu.VMEM((TAIL, NFEAT), jnp.float32),
            pltpu.VMEM_SHARED((N_SC, NFEAT), jnp.float32),
            pltpu.SemaphoreType.DMA,
        ],
    )


_PREC = lax.Precision.HIGHEST


def _spectral(W, u):
    vT = lax.dot_general(u, W, (((1,), (0,)), ((), ())),
                         precision=_PREC)
    v = vT / (jnp.sqrt(jnp.sum(vT * vT)) + SN_EPS)
    Wv = lax.dot_general(v, W, (((1,), (1,)), ((), ())),
                         precision=_PREC)
    u_new = Wv / (jnp.sqrt(jnp.sum(Wv * Wv)) + SN_EPS)
    sigma = jnp.sum(u_new * Wv)
    return W * (1.0 / sigma)


BLK = 2000
GRID = N // BLK


def _tc1_body(x_ref, agg_ref, w1_ref, b1_ref, u1_ref, h_ref, s_ref, q_ref):
    i = pl.program_id(0)
    W1n = _spectral(w1_ref[...], u1_ref[...])
    h0 = x_ref[...] + agg_ref[0] + agg_ref[1]
    h = lax.dot_general(h0, W1n, (((1,), (1,)), ((), ())), precision=_PREC)
    h = jnp.maximum(h + b1_ref[...], 0.0)
    h_ref[...] = h

    @pl.when(i == 0)
    def _():
        s_ref[...] = jnp.zeros_like(s_ref)
        q_ref[...] = jnp.zeros_like(q_ref)

    s_ref[...] += jnp.sum(h, axis=0, keepdims=True)
    q_ref[...] += jnp.sum(h * h, axis=0, keepdims=True)


_tc1 = pl.pallas_call(
    _tc1_body,
    grid=(GRID,),
    in_specs=[
        pl.BlockSpec((BLK, NFEAT), lambda i: (i, 0)),
        pl.BlockSpec((NC, BLK, NFEAT), lambda i: (0, i, 0)),
        pl.BlockSpec((NHID, NFEAT), lambda i: (0, 0)),
        pl.BlockSpec((1, NHID), lambda i: (0, 0)),
        pl.BlockSpec((1, NHID), lambda i: (0, 0)),
    ],
    out_specs=[
        pl.BlockSpec((BLK, NHID), lambda i: (i, 0)),
        pl.BlockSpec((1, NHID), lambda i: (0, 0)),
        pl.BlockSpec((1, NHID), lambda i: (0, 0)),
    ],
    out_shape=[
        jax.ShapeDtypeStruct((N, NHID), jnp.float32),
        jax.ShapeDtypeStruct((1, NHID), jnp.float32),
        jax.ShapeDtypeStruct((1, NHID), jnp.float32),
    ],
)


def _tc2_body(h_ref, s_ref, q_ref, g_ref, be_ref, w2_ref, b2_ref, u2_ref,
              out_ref):
    W2n = _spectral(w2_ref[...], u2_ref[...])
    mean = s_ref[...] * (1.0 / N)
    var = q_ref[...] * (1.0 / N) - mean * mean
    sc = g_ref[...] * lax.rsqrt(var + BN_EPS)
    W2eff = W2n * sc
    cvec = lax.dot_general(be_ref[...] - mean * sc, W2n,
                           (((1,), (1,)), ((), ())), precision=_PREC)
    cvec = cvec + b2_ref[...]
    out = lax.dot_general(h_ref[...], W2eff, (((1,), (1,)), ((), ())),
                          precision=_PREC)
    out_ref[...] = out + cvec


_tc2 = pl.pallas_call(
    _tc2_body,
    grid=(GRID,),
    in_specs=[
        pl.BlockSpec((BLK, NHID), lambda i: (i, 0)),
        pl.BlockSpec((1, NHID), lambda i: (0, 0)),
        pl.BlockSpec((1, NHID), lambda i: (0, 0)),
        pl.BlockSpec((1, NHID), lambda i: (0, 0)),
        pl.BlockSpec((1, NHID), lambda i: (0, 0)),
        pl.BlockSpec((NHID, NHID), lambda i: (0, 0)),
        pl.BlockSpec((1, NHID), lambda i: (0, 0)),
        pl.BlockSpec((1, NHID), lambda i: (0, 0)),
    ],
    out_specs=pl.BlockSpec((BLK, NHID), lambda i: (i, 0)),
    out_shape=jax.ShapeDtypeStruct((N, NHID), jnp.float32),
)


def kernel(x, edge_index, W1, b1, gamma, beta, W2, b2, u1, u2):
    ei_flat = edge_index.reshape(2 * E)

    aggs = _get_sc_aggregate()(x, ei_flat)

    relu_h, sums, sumsq = _tc1(x, aggs, W1, b1.reshape(1, -1),
                               u1.reshape(1, -1))
    out = _tc2(relu_h, sums, sumsq, gamma.reshape(1, -1), beta.reshape(1, -1),
               W2, b2.reshape(1, -1), u2.reshape(1, -1))
    return out


# speedup vs baseline: 2.6510x; 1.0070x over previous
import functools

import jax
import jax.numpy as jnp
from jax import lax
from jax.experimental import pallas as pl
from jax.experimental.pallas import tpu as pltpu
from jax.experimental.pallas import tpu_sc as plsc

N, E, NFEAT, NHID = 10000, 320000, 128, 128
BN_EPS = 1e-5
SN_EPS = 1e-12

NC, NS = 2, 16
NW = NC * NS
CHUNK = 128
EPT = E // NW
NFULL = EPT // CHUNK
TAIL = EPT - NFULL * CHUNK
N_SC = 10240
RPT = N_SC // NS


def _sc_aggregate_body(x_hbm, ei_hbm, out_hbm,
                       srcall, dstv, rows, dstt, rowst, agg, sem):
    c = lax.axis_index("c")
    s = lax.axis_index("s")
    row0 = s * RPT

    zvec = jnp.zeros((16,), jnp.float32)

    def zbody(i, carry):
        rows[i >> 3, pl.ds((i & 7) * 16, 16)] = zvec
        return carry

    lax.fori_loop(0, CHUNK * 8, zbody, 0)
    for kk in range(RPT // CHUNK):
        pltpu.sync_copy(rows, agg.at[pl.ds(row0 + kk * CHUNK, CHUNK)])
    plsc.subcore_barrier()

    wid = s * NC + c
    ebase = wid * EPT

    pltpu.sync_copy(ei_hbm.at[pl.ds(pl.multiple_of(ebase, 16), EPT)], srcall)

    def body(j, carry):
        off = pl.multiple_of(ebase + j * CHUNK, 16)
        pltpu.sync_copy(ei_hbm.at[pl.ds(E + off, CHUNK)], dstv)
        pltpu.async_copy(x_hbm.at[srcall.at[pl.ds(j * CHUNK, CHUNK)]],
                         rows, sem).wait()
        pltpu.sync_copy(rows, agg.at[dstv], add=True)
        return carry

    lax.fori_loop(0, NFULL, body, 0)

    offt = ebase + NFULL * CHUNK
    pltpu.sync_copy(ei_hbm.at[pl.ds(E + offt, TAIL)], dstt)
    pltpu.async_copy(x_hbm.at[srcall.at[pl.ds(NFULL * CHUNK, TAIL)]],
                     rowst, sem).wait()
    pltpu.sync_copy(rowst, agg.at[dstt], add=True)

    plsc.subcore_barrier()

    pltpu.sync_copy(agg.at[pl.ds(row0, RPT)],
                    out_hbm.at[c, pl.ds(row0, RPT)])


@functools.cache
def _get_sc_aggregate():
    mesh = plsc.VectorSubcoreMesh(core_axis_name="c", subcore_axis_name="s",
                                  num_cores=NC, num_subcores=NS)
    return pl.kernel(
        _sc_aggregate_body,
        out_type=jax.ShapeDtypeStruct((NC, N_SC, NFEAT), jnp.float32),
        mesh=mesh,
        scratch_types=[
            pltpu.VMEM((EPT,), jnp.int32),
            pltpu.VMEM((CHUNK,), jnp.int32),
            pltpu.VMEM((CHUNK, NFEAT), jnp.float32),
            pltpu.VMEM((TAIL,), jnp.int32),
            pltpu.VMEM((TAIL, NFEAT), jnp.float32),
            pltpu.VMEM_SHARED((N_SC, NFEAT), jnp.float32),
            pltpu.SemaphoreType.DMA,
        ],
    )


_PREC = lax.Precision.HIGHEST


def _spectral(W, u):
    vT = lax.dot_general(u, W, (((1,), (0,)), ((), ())),
                         precision=_PREC)
    v = vT / (jnp.sqrt(jnp.sum(vT * vT)) + SN_EPS)
    Wv = lax.dot_general(v, W, (((1,), (1,)), ((), ())),
                         precision=_PREC)
    u_new = Wv / (jnp.sqrt(jnp.sum(Wv * Wv)) + SN_EPS)
    sigma = jnp.sum(u_new * Wv)
    return W * (1.0 / sigma)


BLK = 2000
GRID = N // BLK


def _tc_body(x_ref, agg_ref, w1_ref, b1_ref, u1_ref, g_ref, be_ref,
             w2_ref, b2_ref, u2_ref, out_ref, hbuf, sbuf, qbuf):
    i = pl.program_id(0)

    @pl.when(i < GRID)
    def _():
        W1n = _spectral(w1_ref[...], u1_ref[...])
        h0 = x_ref[...] + agg_ref[0] + agg_ref[1]
        h = lax.dot_general(h0, W1n, (((1,), (1,)), ((), ())),
                            precision=_PREC)
        h = jnp.maximum(h + b1_ref[...], 0.0)
        hbuf[pl.ds(i * BLK, BLK), :] = h

        @pl.when(i == 0)
        def _():
            sbuf[...] = jnp.zeros_like(sbuf)
            qbuf[...] = jnp.zeros_like(qbuf)

        sbuf[...] += jnp.sum(h, axis=0, keepdims=True)
        qbuf[...] += jnp.sum(h * h, axis=0, keepdims=True)

    @pl.when(i >= GRID)
    def _():
        ib = i - GRID
        W2n = _spectral(w2_ref[...], u2_ref[...])
        mean = sbuf[...] * (1.0 / N)
        var = qbuf[...] * (1.0 / N) - mean * mean
        sc = g_ref[...] * lax.rsqrt(var + BN_EPS)
        W2eff = W2n * sc
        cvec = lax.dot_general(be_ref[...] - mean * sc, W2n,
                               (((1,), (1,)), ((), ())), precision=_PREC)
        cvec = cvec + b2_ref[...]
        h = hbuf[pl.ds(ib * BLK, BLK), :]
        out = lax.dot_general(h, W2eff, (((1,), (1,)), ((), ())),
                              precision=_PREC)
        out_ref[...] = out + cvec


def _make_tc(interpret=False):
    vec = pl.BlockSpec((1, NHID), lambda i: (0, 0))
    return pl.pallas_call(
        _tc_body,
        grid=(2 * GRID,),
        in_specs=[
            pl.BlockSpec((BLK, NFEAT),
                         lambda i: (jnp.where(i < GRID, i, GRID - 1), 0)),
            pl.BlockSpec((NC, BLK, NFEAT),
                         lambda i: (0, jnp.where(i < GRID, i, GRID - 1), 0)),
            pl.BlockSpec((NHID, NFEAT), lambda i: (0, 0)),
            vec, vec, vec, vec,
            pl.BlockSpec((NHID, NHID), lambda i: (0, 0)),
            vec, vec,
        ],
        out_specs=pl.BlockSpec((BLK, NHID),
                               lambda i: (jnp.where(i < GRID, 0, i - GRID), 0)),
        out_shape=jax.ShapeDtypeStruct((N, NHID), jnp.float32),
        scratch_shapes=[
            pltpu.VMEM((N, NHID), jnp.float32),
            pltpu.VMEM((1, NHID), jnp.float32),
            pltpu.VMEM((1, NHID), jnp.float32),
        ],
        interpret=interpret,
    )


_tc = _make_tc()


def kernel(x, edge_index, W1, b1, gamma, beta, W2, b2, u1, u2):
    ei_flat = edge_index.reshape(2 * E)

    aggs = _get_sc_aggregate()(x, ei_flat)

    out = _tc(x, aggs, W1, b1.reshape(1, -1), u1.reshape(1, -1),
              gamma.reshape(1, -1), beta.reshape(1, -1),
              W2, b2.reshape(1, -1), u2.reshape(1, -1))
    return out
